# baseline (speedup 1.0000x reference)
import functools

import jax
import jax.numpy as jnp
from jax import lax
from jax.experimental import pallas as pl
from jax.experimental.pallas import tpu as pltpu
from jax.experimental.pallas import tpu_sc as plsc

N_NODES = 10000
N_EDGES = 320000
EPK = N_EDGES // 4
NPK = N_NODES // 4
EDGE_CHUNK = 4000
S2S_EDGE_CHUNK = 10000

SC_CORES = 2
SC_TILES = 16
SC_WORKERS = SC_CORES * SC_TILES
GATHER_K = 1000
SCATTER_K = 1000

BF = jnp.bfloat16


def _lrelu(x):
    return jnp.where(x > 0, x, 0.01 * x)


def _sigm(x):
    return 1.0 / (1.0 + jnp.exp(-x))


def _tanh(x):
    e2 = jnp.exp(-2.0 * x)
    return (1.0 - e2) / (1.0 + e2)


def _mm(x, w):
    return jnp.matmul(x, w, preferred_element_type=jnp.float32)


def _bd4(w):
    return jnp.kron(jnp.eye(4, dtype=w.dtype), w)


def _tile4(b):
    return jnp.tile(b, (4,))[None, :]


def _fold4(d):
    ii = jnp.arange(4 * d)
    return (ii[:, None] % d == jnp.arange(d)[None, :]).astype(jnp.float32)


def _sc_mesh():
    return plsc.VectorSubcoreMesh(core_axis_name="c", subcore_axis_name="s")


def _gather_call(h_bf, src, dst):
    n2 = h_bf.shape[1]
    epw = N_EDGES // SC_WORKERS
    k = GATHER_K
    niter = epw // k

    @functools.partial(
        pl.kernel,
        out_type=[jax.ShapeDtypeStruct((N_EDGES, n2), BF),
                  jax.ShapeDtypeStruct((N_EDGES, n2), BF)],
        mesh=_sc_mesh(),
        scratch_types=[pltpu.VMEM((k,), jnp.int32),
                       pltpu.VMEM((k,), jnp.int32),
                       pltpu.VMEM((k, n2), BF),
                       pltpu.VMEM((k, n2), BF),
                       pltpu.SemaphoreType.DMA,
                       pltpu.SemaphoreType.DMA],
        compiler_params=pltpu.CompilerParams(use_tc_tiling_on_sc=False),
    )
    def gk(h_hbm, src_hbm, dst_hbm, a_hbm, b_hbm,
           idx_s, idx_d, rows_a, rows_b, sem_a, sem_b):
        wid = lax.axis_index("s") * SC_CORES + lax.axis_index("c")
        base = wid * epw

        def body(t, carry):
            off = base + t * k
            pltpu.sync_copy(src_hbm.at[pl.ds(off, k)], idx_s)
            pltpu.sync_copy(dst_hbm.at[pl.ds(off, k)], idx_d)
            ca = pltpu.async_copy(h_hbm.at[idx_s], rows_a, sem_a)
            cb = pltpu.async_copy(h_hbm.at[idx_d], rows_b, sem_b)
            ca.wait()
            cb.wait()
            pltpu.sync_copy(rows_a, a_hbm.at[pl.ds(off, k)])
            pltpu.sync_copy(rows_b, b_hbm.at[pl.ds(off, k)])
            return carry

        lax.fori_loop(0, niter, body, 0)

    return gk(h_bf, src, dst)


def _scatter_call(e_new, dst, zeros_tile, ones_k, with_counts):
    n2 = e_new.shape[1]
    epc = N_EDGES // SC_CORES
    epw = epc // SC_TILES
    k = SCATTER_K
    niter = epw // k
    stripe = N_NODES // SC_TILES

    out_type = [jax.ShapeDtypeStruct((SC_CORES * N_NODES, n2), jnp.float32)]
    scratch = [pltpu.VMEM((k,), jnp.int32),
               pltpu.VMEM((k, n2), jnp.float32),
               pltpu.VMEM_SHARED((N_NODES, n2), jnp.float32)]
    if with_counts:
        out_type.append(jax.ShapeDtypeStruct((SC_CORES * N_NODES, n2),
                                             jnp.float32))
        scratch.append(pltpu.VMEM_SHARED((N_NODES, n2), jnp.float32))
        scratch.append(pltpu.VMEM((k, n2), jnp.float32))

    @functools.partial(
        pl.kernel,
        out_type=out_type,
        mesh=_sc_mesh(),
        scratch_types=scratch,
        compiler_params=pltpu.CompilerParams(use_tc_tiling_on_sc=False),
    )
    def sk(e_hbm, dst_hbm, z_hbm, o_hbm, *rest):
        if with_counts:
            (p_hbm, c_hbm, idx_v, rows_v, shared, shared_cnt, ones_v) = rest
        else:
            (p_hbm, idx_v, rows_v, shared) = rest
        cid = lax.axis_index("c")
        sid = lax.axis_index("s")
        row0 = sid * stripe
        pltpu.sync_copy(z_hbm, shared.at[pl.ds(row0, stripe)])
        if with_counts:
            pltpu.sync_copy(z_hbm, shared_cnt.at[pl.ds(row0, stripe)])
            pltpu.sync_copy(o_hbm, ones_v)
        plsc.subcore_barrier()
        base = cid * epc + sid * epw

        def body(t, carry):
            off = base + t * k
            pltpu.sync_copy(dst_hbm.at[pl.ds(off, k)], idx_v)
            pltpu.sync_copy(e_hbm.at[pl.ds(off, k)], rows_v)
            pltpu.sync_copy(rows_v, shared.at[idx_v], add=True)
            if with_counts:
                pltpu.sync_copy(ones_v, shared_cnt.at[idx_v], add=True)
            return carry

        lax.fori_loop(0, niter, body, 0)
        plsc.subcore_barrier()
        obase = cid * N_NODES + row0
        pltpu.sync_copy(shared.at[pl.ds(row0, stripe)],
                        p_hbm.at[pl.ds(obase, stripe)])
        if with_counts:
            pltpu.sync_copy(shared_cnt.at[pl.ds(row0, stripe)],
                            c_hbm.at[pl.ds(obase, stripe)])

    return sk(e_new, dst, zeros_tile, ones_k)


def _emb_body(nf_ref, wn_ref, bn_ref, gf_ref, wg_ref, bg_ref,
              h_ref, hbf_ref, u_ref):
    h = nf_ref[...] @ wn_ref[...] + bn_ref[...]
    h_ref[...] = h
    hbf_ref[...] = h.astype(BF)
    u_ref[...] = gf_ref[...] @ wg_ref[...] + bg_ref[...]


def _emb_call(node_feats, graph_feats, params):
    wn = params["node_emb"]["w"]
    bn = params["node_emb"]["b"][None, :]
    wg = params["graph_emb"]["w"]
    bg = params["graph_emb"]["b"][None, :]
    n2 = wn.shape[1]
    return pl.pallas_call(
        _emb_body,
        out_shape=[
            jax.ShapeDtypeStruct((N_NODES, n2), jnp.float32),
            jax.ShapeDtypeStruct((N_NODES, n2), BF),
            jax.ShapeDtypeStruct((1, n2), jnp.float32),
        ],
    )(node_feats, wn, bn, graph_feats, wg, bg)


def _edge_body(emb, lastblk,
               a_ref, b_ref, e_ref, u_ref,
               wa_ref, wb_ref, we_ref, w1u_ref, b1_ref,
               w2_ref, b2_ref, w3_ref, b3_ref, fold_ref,
               *rest):
    idx = 0
    if emb:
        ksel_ref, bemb_ref = rest[idx], rest[idx + 1]
        idx += 2
    outs = rest[idx:]
    enew_ref, ecs_ref = outs[0], outs[1]
    outs = outs[2:]
    if lastblk:
        ebf_ref = outs[0]
        outs = outs[1:]
    acc_ref = outs[0]

    j = pl.program_id(0)
    nj = pl.num_programs(0)
    a = a_ref[...]
    b = b_ref[...]
    if emb:
        e = _mm(e_ref[...].astype(BF), ksel_ref[0]) + bemb_ref[...]
    else:
        e = e_ref[...]
    ebf = e.astype(BF)
    ub = u_ref[...] @ w1u_ref[...] + b1_ref[...]
    ubt = jnp.concatenate([ub, ub, ub, ub], axis=1)
    x1 = _lrelu(_mm(a, wa_ref[...]) + _mm(b, wb_ref[...])
                + _mm(ebf, we_ref[...]) + ubt)
    x2 = _lrelu(_mm(x1.astype(BF), w2_ref[...]) + b2_ref[...])
    en = _mm(x2.astype(BF), w3_ref[...]) + b3_ref[...] + e
    enew_ref[...] = en
    if lastblk:
        ebf_ref[...] = en.astype(BF)
    cs = jnp.sum(en, axis=0, keepdims=True)

    @pl.when(j == 0)
    def _():
        acc_ref[...] = cs

    @pl.when(j > 0)
    def _():
        acc_ref[...] = acc_ref[...] + cs

    @pl.when(j == nj - 1)
    def _():
        ecs_ref[...] = acc_ref[...] @ fold_ref[...]


def _edge_call(a, b, e_in, u, blk, emb_w, lastblk):
    phi = blk["phi_e"]
    w1full = phi[0]["w"]
    wa = _bd4(w1full[:32]).astype(BF)
    wb = _bd4(w1full[32:64]).astype(BF)
    we = _bd4(w1full[64:96]).astype(BF)
    w1u = w1full[96:]
    b1 = phi[0]["b"][None, :]
    w2 = _bd4(phi[1]["w"]).astype(BF)
    b2 = _tile4(phi[1]["b"])
    w3 = _bd4(phi[2]["w"]).astype(BF)
    b3 = _tile4(phi[2]["b"])
    fold = _fold4(32)

    C = EDGE_CHUNK
    nh = (EPK // C) // 2
    emb = emb_w is not None

    def row_spec(d, dt=None):
        return pl.BlockSpec((C, d), lambda j: (j, 0))

    def full_spec(shape):
        return pl.BlockSpec(shape, lambda j: (0,) * len(shape))

    if emb:
        e_spec = pl.BlockSpec(
            (C, 128), lambda j: (jnp.where(j < nh, j, j - nh), 0))
    else:
        e_spec = row_spec(128)

    in_specs = [row_spec(128), row_spec(128), e_spec,
                full_spec(u.shape),
                full_spec(wa.shape), full_spec(wb.shape), full_spec(we.shape),
                full_spec(w1u.shape), full_spec(b1.shape),
                full_spec(w2.shape), full_spec(b2.shape),
                full_spec(w3.shape), full_spec(b3.shape),
                full_spec(fold.shape)]
    args = [a, b, e_in, u, wa, wb, we, w1u, b1, w2, b2, w3, b3, fold]
    if emb:
        ksel, bemb = emb_w
        in_specs += [
            pl.BlockSpec((1, 128, 128),
                         lambda j: (jnp.where(j < nh, 0, 1), 0, 0)),
            full_spec(bemb.shape)]
        args += [ksel, bemb]

    out_specs = [row_spec(128), pl.BlockSpec((1, 32), lambda j: (0, 0))]
    out_shape = [jax.ShapeDtypeStruct((EPK, 128), jnp.float32),
                 jax.ShapeDtypeStruct((1, 32), jnp.float32)]
    if lastblk:
        out_specs.append(row_spec(128))
        out_shape.append(jax.ShapeDtypeStruct((EPK, 128), BF))

    return pl.pallas_call(
        functools.partial(_edge_body, emb, lastblk),
        grid=(EPK // C,),
        in_specs=in_specs,
        out_specs=out_specs,
        out_shape=out_shape,
        scratch_shapes=[pltpu.VMEM((1, 128), jnp.float32)],
        compiler_params=pltpu.CompilerParams(
            dimension_semantics=("arbitrary",)),
    )(*args)


def _node_body(firstblk, lastblk,
               h_ref, p0_ref, p1_ref, cnt_ref, u_ref, ecs_ref,
               wv1h_ref, wv1a_ref, wv1u_ref, bv1_ref,
               wv2_ref, bv2_ref, wv3_ref, bv3_ref, fold_ref,
               wu1h_ref, wu1e_ref, wu1u_ref, bu1_ref,
               wu2_ref, bu2_ref, wu3_ref, bu3_ref,
               *rest):
    idx = 0
    if firstblk:
        c1_ref = rest[idx]
        idx += 1
    if lastblk:
        wnl_ref, bnl_ref = rest[idx], rest[idx + 1]
        idx += 2
    outs = rest[idx:]
    hn_ref, hbf_ref, un_ref = outs[0], outs[1], outs[2]
    outs = outs[3:]
    if firstblk:
        cnt_out_ref = outs[0]
        outs = outs[1:]
    if lastblk:
        h16_ref = outs[0]

    h = h_ref[...]
    if firstblk:
        cnt = cnt_ref[...] + c1_ref[...]
        cnt_out_ref[...] = cnt
    else:
        cnt = cnt_ref[...]
    agg = (p0_ref[...] + p1_ref[...]) / jnp.maximum(cnt, 1.0)
    urow = u_ref[...]
    ub = urow @ wv1u_ref[...] + bv1_ref[...]
    ubt = jnp.concatenate([ub, ub, ub, ub], axis=1)
    x1 = _lrelu(_mm(h.astype(BF), wv1h_ref[...])
                + _mm(agg.astype(BF), wv1a_ref[...]) + ubt)
    x2 = _lrelu(_mm(x1.astype(BF), wv2_ref[...]) + bv2_ref[...])
    hn = _mm(x2.astype(BF), wv3_ref[...]) + bv3_ref[...] + h
    hn_ref[...] = hn
    hbf_ref[...] = hn.astype(BF)

    hm = (jnp.sum(hn, axis=0, keepdims=True) @ fold_ref[...]) * (1.0 / N_NODES)
    em = ecs_ref[...] * (1.0 / N_EDGES)
    y1 = _lrelu(hm @ wu1h_ref[...] + em @ wu1e_ref[...]
                + urow @ wu1u_ref[...] + bu1_ref[...])
    y2 = _lrelu(y1 @ wu2_ref[...] + bu2_ref[...])
    un_ref[...] = y2 @ wu3_ref[...] + bu3_ref[...] + urow

    if lastblk:
        h16_ref[...] = hn @ wnl_ref[...] + bnl_ref[...]


def _node_call(h, p0, p1, cnt, c1, u, ecs, blk, params, firstblk, lastblk):
    phiv = blk["phi_v"]
    wv1full = phiv[0]["w"]
    wv1h = _bd4(wv1full[:32]).astype(BF)
    wv1a = _bd4(wv1full[32:64]).astype(BF)
    wv1u = wv1full[64:]
    bv1 = phiv[0]["b"][None, :]
    wv2 = _bd4(phiv[1]["w"]).astype(BF)
    bv2 = _tile4(phiv[1]["b"])
    wv3 = _bd4(phiv[2]["w"]).astype(BF)
    bv3 = _tile4(phiv[2]["b"])
    fold = _fold4(32)
    phiu = blk["phi_u"]
    wu1full = phiu[0]["w"]
    wu1h = wu1full[:32]
    wu1e = wu1full[32:64]
    wu1u = wu1full[64:]
    bu1 = phiu[0]["b"][None, :]
    wu2, bu2 = phiu[1]["w"], phiu[1]["b"][None, :]
    wu3, bu3 = phiu[2]["w"], phiu[2]["b"][None, :]

    args = [h, p0, p1, cnt, u, ecs,
            wv1h, wv1a, wv1u, bv1, wv2, bv2, wv3, bv3, fold,
            wu1h, wu1e, wu1u, bu1, wu2, bu2, wu3, bu3]
    out_shape = [jax.ShapeDtypeStruct((NPK, 128), jnp.float32),
                 jax.ShapeDtypeStruct((NPK, 128), BF),
                 jax.ShapeDtypeStruct((1, 32), jnp.float32)]
    if firstblk:
        args += [c1]
        out_shape.append(jax.ShapeDtypeStruct((NPK, 128), jnp.float32))
    if lastblk:
        wnl = _bd4(params["node_last"]["w"])
        bnl = _tile4(params["node_last"]["b"])
        args += [wnl, bnl]
        out_shape.append(jax.ShapeDtypeStruct((NPK, 64), jnp.float32))

    return pl.pallas_call(
        functools.partial(_node_body, firstblk, lastblk),
        out_shape=out_shape,
    )(*args)


def _s2s_body(nchunks, has_proj, D,
              *refs):
    if has_proj:
        (x_ref, welt_ref, wel_ref, bel_ref, bd_ref, bdt_ref, fold_ref,
         wih0_ref, whh0_ref, b0_ref, wih1_ref, whh1_ref, b1_ref,
         out_ref, st_ref) = refs
    else:
        (x_ref, bd_ref, bdt_ref, fold_ref,
         wih0_ref, whh0_ref, b0_ref, wih1_ref, whh1_ref, b1_ref,
         out_ref, st_ref) = refs
    i = pl.program_id(0)
    j = pl.program_id(1)
    L = 4 * D

    @pl.when(jnp.logical_and(i == 0, j == 0))
    def _():
        st_ref[...] = jnp.zeros((8, 128), jnp.float32)

    @pl.when(j == 0)
    def _():
        qs = st_ref[4:5, 0:32]
        h0 = st_ref[0:1, 0:16]
        c0 = st_ref[1:2, 0:16]
        h1 = st_ref[2:3, 0:16]
        c1 = st_ref[3:4, 0:16]
        g = qs @ wih0_ref[...] + h0 @ whh0_ref[...] + b0_ref[...]
        ig = _sigm(g[:, 0:16])
        fg = _sigm(g[:, 16:32])
        gg = _tanh(g[:, 32:48])
        og = _sigm(g[:, 48:64])
        c0n = fg * c0 + ig * gg
        h0n = og * _tanh(c0n)
        g2 = h0n @ wih1_ref[...] + h1 @ whh1_ref[...] + b1_ref[...]
        ig2 = _sigm(g2[:, 0:16])
        fg2 = _sigm(g2[:, 16:32])
        gg2 = _tanh(g2[:, 32:48])
        og2 = _sigm(g2[:, 48:64])
        c1n = fg2 * c1 + ig2 * gg2
        h1n = og2 * _tanh(c1n)
        st_ref[0:1, 0:16] = h0n
        st_ref[1:2, 0:16] = c0n
        st_ref[2:3, 0:16] = h1n
        st_ref[3:4, 0:16] = c1n
        st_ref[5:6, 0:16] = h1n
        st_ref[6:7, :] = jnp.zeros((1, 128), jnp.float32)
        st_ref[7:8, 0:1] = jnp.full((1, 1), -1e30, jnp.float32)
        st_ref[7:8, 1:2] = jnp.zeros((1, 1), jnp.float32)

    x = x_ref[...]
    q = st_ref[5:6, 0:16]
    if has_proj:
        qw = q @ welt_ref[...]
    else:
        qw = q
    qwl = jnp.concatenate([qw, qw, qw, qw], axis=1).astype(BF)
    xb = x if x.dtype == BF else x.astype(BF)
    sc4 = _mm(xb * qwl, bd_ref[...])
    m_old = st_ref[7:8, 0:1]
    s_old = st_ref[7:8, 1:2]
    cmax = jnp.max(sc4, axis=(0, 1), keepdims=True)
    m_new = jnp.maximum(m_old, cmax)
    scale = jnp.exp(m_old - m_new)
    w4 = jnp.exp(sc4 - m_new)
    ssum = jnp.sum(w4, axis=(0, 1), keepdims=True)
    wx = _mm(w4.astype(BF), bdt_ref[...])
    v = jnp.sum(wx * x.astype(jnp.float32), axis=0, keepdims=True)
    st_ref[6:7, 0:L] = st_ref[6:7, 0:L] * scale + v
    st_ref[7:8, 0:1] = m_new
    st_ref[7:8, 1:2] = s_old * scale + ssum

    @pl.when(j == nchunks - 1)
    def _():
        vd = (st_ref[6:7, 0:L] @ fold_ref[...]) / st_ref[7:8, 1:2]
        if has_proj:
            r = vd @ wel_ref[...] + bel_ref[...]
        else:
            r = vd
        qsn = jnp.concatenate([st_ref[5:6, 0:16], r], axis=1)
        st_ref[4:5, 0:32] = qsn

        @pl.when(i == 2)
        def _():
            out_ref[...] = qsn


def _s2s_call(x_packed, p, chunk, proj):
    R = x_packed.shape[0]
    L = x_packed.shape[1]
    D = L // 4
    nchunks = R // chunk
    ii = jnp.arange(L)
    bd = (ii[:, None] // D == jnp.arange(4)[None, :]).astype(BF)
    bdt = bd.T
    fold = (ii[:, None] % D == jnp.arange(D)[None, :]).astype(jnp.float32)
    wih0, whh0, b0 = p["wih0"], p["whh0"], p["b0"][None, :]
    wih1, whh1, b1 = p["wih1"], p["whh1"], p["b1"][None, :]

    def full_spec2(shape):
        return pl.BlockSpec(shape, lambda i, j: (0,) * len(shape))

    in_specs = [pl.BlockSpec((chunk, L), lambda i, j: (j, 0))]
    args = [x_packed]
    if proj is not None:
        wel, bel = proj
        welt = wel.T
        in_specs += [full_spec2(welt.shape), full_spec2(wel.shape),
                     full_spec2(bel.shape)]
        args += [welt, wel, bel]
    in_specs += [full_spec2(bd.shape), full_spec2(bdt.shape),
                 full_spec2(fold.shape),
                 full_spec2(wih0.shape), full_spec2(whh0.shape),
                 full_spec2(b0.shape),
                 full_spec2(wih1.shape), full_spec2(whh1.shape),
                 full_spec2(b1.shape)]
    args += [bd, bdt, fold, wih0, whh0, b0, wih1, whh1, b1]

    return pl.pallas_call(
        functools.partial(_s2s_body, nchunks, proj is not None, D),
        grid=(3, nchunks),
        in_specs=in_specs,
        out_specs=pl.BlockSpec((1, 32), lambda i, j: (0, 0)),
        out_shape=jax.ShapeDtypeStruct((1, 32), jnp.float32),
        scratch_shapes=[pltpu.VMEM((8, 128), jnp.float32)],
        compiler_params=pltpu.CompilerParams(
            dimension_semantics=("arbitrary", "arbitrary")),
    )(*args)


def _pred_body(hp_ref, ep_ref, w1h_ref, w1e_ref, b1_ref, w2_ref, b2_ref,
               out_ref):
    y1 = _lrelu(hp_ref[...] @ w1h_ref[...] + ep_ref[...] @ w1e_ref[...]
                + b1_ref[...])
    out_ref[...] = y1 @ w2_ref[...] + b2_ref[...]


def _pred_call(hp, ep, p):
    w1full = p[0]["w"]
    w1h = w1full[:32]
    w1e = w1full[32:]
    b1 = p[0]["b"][None, :]
    w2, b2 = p[1]["w"], p[1]["b"][None, :]
    return pl.pallas_call(
        _pred_body,
        out_shape=jax.ShapeDtypeStruct((1, 1), jnp.float32),
    )(hp, ep, w1h, w1e, b1, w2, b2)


def kernel(node_feats, edge_feats, graph_feats, edge_index, params):
    ei8 = edge_index.reshape(2, N_EDGES // 8, 8)
    src = jnp.concatenate([ei8[0, :, :4].reshape(-1),
                           ei8[0, :, 4:].reshape(-1)])
    dst = jnp.concatenate([ei8[1, :, :4].reshape(-1),
                           ei8[1, :, 4:].reshape(-1)])
    ef8 = jnp.reshape(edge_feats, (N_EDGES // 8, 128))

    h32, hbf32, u = _emb_call(node_feats, graph_feats, params)

    wemb = params["edge_emb"]["w"]
    kemb = jnp.kron(jnp.eye(4, dtype=wemb.dtype), wemb)
    z64 = jnp.zeros((64, 128), jnp.float32)
    ksel = jnp.stack([jnp.concatenate([kemb, z64], axis=0),
                      jnp.concatenate([z64, kemb], axis=0)]).astype(BF)
    bemb = _tile4(params["edge_emb"]["b"])

    zeros_tile = jnp.zeros((N_NODES // SC_TILES, 32), jnp.float32)
    ones_k = jnp.ones((SCATTER_K, 32), jnp.float32)

    cnt_pk = None
    h16 = None
    e_in = ef8
    e_bf = None
    for bi, blk in enumerate(params["blocks"]):
        a32, b32 = _gather_call(hbf32, src, dst)
        a_pk = jnp.reshape(a32, (EPK, 128))
        b_pk = jnp.reshape(b32, (EPK, 128))
        firstblk = bi == 0
        lastblk = bi == 2
        h_pk = jnp.reshape(h32, (NPK, 128))
        emb_w = (ksel, bemb) if firstblk else None
        eouts = _edge_call(a_pk, b_pk, e_in, u, blk, emb_w, lastblk)
        enew_pk, ecs = eouts[0], eouts[1]
        if lastblk:
            e_bf = eouts[2]
        souts = _scatter_call(jnp.reshape(enew_pk, (N_EDGES, 32)), dst,
                              zeros_tile, ones_k, firstblk)
        parts = jnp.reshape(souts[0], (2 * NPK, 128))
        p0 = parts[:NPK]
        p1 = parts[NPK:]
        if firstblk:
            cparts = jnp.reshape(souts[1], (2 * NPK, 128))
            c0 = cparts[:NPK]
            c1 = cparts[NPK:]
        else:
            c0, c1 = cnt_pk, None
        nouts = _node_call(h_pk, p0, p1, c0, c1, u, ecs, blk, params,
                           firstblk, lastblk)
        hn_pk, hn_bf, u = nouts[0], nouts[1], nouts[2]
        nouts = nouts[3:]
        if firstblk:
            cnt_pk = nouts[0]
            nouts = nouts[1:]
        if lastblk:
            h16 = nouts[0]
        h32 = jnp.reshape(hn_pk, (N_NODES, 32))
        hbf32 = jnp.reshape(hn_bf, (N_NODES, 32))
        e_in = enew_pk

    wel = params["edge_last"]["w"]
    bel = params["edge_last"]["b"][None, :]
    hp = _s2s_call(h16, params["s2s_node"], NPK, None)
    ep = _s2s_call(e_bf, params["s2s_edge"], S2S_EDGE_CHUNK, (wel, bel))
    return _pred_call(hp, ep, params["pred"])


# speedup vs baseline: 1.4220x; 1.4220x over previous
import functools

import jax
import jax.numpy as jnp
from jax import lax
from jax.experimental import pallas as pl
from jax.experimental.pallas import tpu as pltpu
from jax.experimental.pallas import tpu_sc as plsc

N_NODES = 10000
N_EDGES = 320000
EPK = N_EDGES // 4
NPK = N_NODES // 4
EDGE_CHUNK = 4000
S2S_EDGE_CHUNK = 10000

SC_CORES = 2
SC_TILES = 16
SC_WORKERS = SC_CORES * SC_TILES
GATHER_K = 1000
SCATTER_K = 1000

BF = jnp.bfloat16


def _lrelu(x):
    return jnp.where(x > 0, x, 0.01 * x)


def _sigm(x):
    return 1.0 / (1.0 + jnp.exp(-x))


def _tanh(x):
    e2 = jnp.exp(-2.0 * x)
    return (1.0 - e2) / (1.0 + e2)


def _mm(x, w):
    return jnp.matmul(x, w, preferred_element_type=jnp.float32)


def _bd4(w):
    return jnp.kron(jnp.eye(4, dtype=w.dtype), w)


def _tile4(b):
    return jnp.tile(b, (4,))[None, :]


def _fold4(d):
    ii = jnp.arange(4 * d)
    return (ii[:, None] % d == jnp.arange(d)[None, :]).astype(jnp.float32)


def _sc_mesh():
    return plsc.VectorSubcoreMesh(core_axis_name="c", subcore_axis_name="s")


def _gather_call(h, src, dst):
    n2 = h.shape[1]
    epw = N_EDGES // SC_WORKERS
    k = GATHER_K
    niter = epw // k

    @functools.partial(
        pl.kernel,
        out_type=[jax.ShapeDtypeStruct((N_EDGES, n2), jnp.float32),
                  jax.ShapeDtypeStruct((N_EDGES, n2), jnp.float32)],
        mesh=_sc_mesh(),
        scratch_types=[pltpu.VMEM((k,), jnp.int32),
                       pltpu.VMEM((k,), jnp.int32),
                       pltpu.VMEM((k, n2), jnp.float32),
                       pltpu.VMEM((k, n2), jnp.float32),
                       pltpu.SemaphoreType.DMA,
                       pltpu.SemaphoreType.DMA],
        compiler_params=pltpu.CompilerParams(use_tc_tiling_on_sc=False),
    )
    def gk(h_hbm, src_hbm, dst_hbm, a_hbm, b_hbm,
           idx_s, idx_d, rows_a, rows_b, sem_a, sem_b):
        wid = lax.axis_index("s") * SC_CORES + lax.axis_index("c")
        base = wid * epw

        def body(t, carry):
            off = base + t * k
            pltpu.sync_copy(src_hbm.at[pl.ds(off, k)], idx_s)
            pltpu.sync_copy(dst_hbm.at[pl.ds(off, k)], idx_d)
            ca = pltpu.async_copy(h_hbm.at[idx_s], rows_a, sem_a)
            cb = pltpu.async_copy(h_hbm.at[idx_d], rows_b, sem_b)
            ca.wait()
            cb.wait()
            pltpu.sync_copy(rows_a, a_hbm.at[pl.ds(off, k)])
            pltpu.sync_copy(rows_b, b_hbm.at[pl.ds(off, k)])
            return carry

        lax.fori_loop(0, niter, body, 0)

    return gk(h, src, dst)


def _scatter_call(e_new, dst, zeros_tile, ones_k, with_counts):
    n2 = e_new.shape[1]
    epc = N_EDGES // SC_CORES
    epw = epc // SC_TILES
    k = SCATTER_K
    niter = epw // k
    stripe = N_NODES // SC_TILES

    out_type = [jax.ShapeDtypeStruct((SC_CORES * N_NODES, n2), jnp.float32)]
    scratch = [pltpu.VMEM((k,), jnp.int32),
               pltpu.VMEM((k, n2), jnp.float32),
               pltpu.VMEM_SHARED((N_NODES, n2), jnp.float32)]
    if with_counts:
        out_type.append(jax.ShapeDtypeStruct((SC_CORES * N_NODES, n2),
                                             jnp.float32))
        scratch.append(pltpu.VMEM_SHARED((N_NODES, n2), jnp.float32))
        scratch.append(pltpu.VMEM((k, n2), jnp.float32))

    @functools.partial(
        pl.kernel,
        out_type=out_type,
        mesh=_sc_mesh(),
        scratch_types=scratch,
        compiler_params=pltpu.CompilerParams(use_tc_tiling_on_sc=False),
    )
    def sk(e_hbm, dst_hbm, z_hbm, o_hbm, *rest):
        if with_counts:
            (p_hbm, c_hbm, idx_v, rows_v, shared, shared_cnt, ones_v) = rest
        else:
            (p_hbm, idx_v, rows_v, shared) = rest
        cid = lax.axis_index("c")
        sid = lax.axis_index("s")
        row0 = sid * stripe
        pltpu.sync_copy(z_hbm, shared.at[pl.ds(row0, stripe)])
        if with_counts:
            pltpu.sync_copy(z_hbm, shared_cnt.at[pl.ds(row0, stripe)])
            pltpu.sync_copy(o_hbm, ones_v)
        plsc.subcore_barrier()
        base = cid * epc + sid * epw

        def body(t, carry):
            off = base + t * k
            pltpu.sync_copy(dst_hbm.at[pl.ds(off, k)], idx_v)
            pltpu.sync_copy(e_hbm.at[pl.ds(off, k)], rows_v)
            pltpu.sync_copy(rows_v, shared.at[idx_v], add=True)
            if with_counts:
                pltpu.sync_copy(ones_v, shared_cnt.at[idx_v], add=True)
            return carry

        lax.fori_loop(0, niter, body, 0)
        plsc.subcore_barrier()
        obase = cid * N_NODES + row0
        pltpu.sync_copy(shared.at[pl.ds(row0, stripe)],
                        p_hbm.at[pl.ds(obase, stripe)])
        if with_counts:
            pltpu.sync_copy(shared_cnt.at[pl.ds(row0, stripe)],
                            c_hbm.at[pl.ds(obase, stripe)])

    return sk(e_new, dst, zeros_tile, ones_k)


def _emb_body(nf_ref, wn_ref, bn_ref, gf_ref, wg_ref, bg_ref,
              h_ref, u_ref):
    h_ref[...] = nf_ref[...] @ wn_ref[...] + bn_ref[...]
    u_ref[...] = gf_ref[...] @ wg_ref[...] + bg_ref[...]


def _emb_call(node_feats, graph_feats, params):
    wn = params["node_emb"]["w"]
    bn = params["node_emb"]["b"][None, :]
    wg = params["graph_emb"]["w"]
    bg = params["graph_emb"]["b"][None, :]
    n2 = wn.shape[1]
    return pl.pallas_call(
        _emb_body,
        out_shape=[
            jax.ShapeDtypeStruct((N_NODES, n2), jnp.float32),
            jax.ShapeDtypeStruct((1, n2), jnp.float32),
        ],
    )(node_feats, wn, bn, graph_feats, wg, bg)


def _edge_body(emb, lastblk,
               a_ref, b_ref, e_ref, u_ref,
               wa_ref, wb_ref, we_ref, w1u_ref, b1_ref,
               w2_ref, b2_ref, w3_ref, b3_ref, fold_ref,
               *rest):
    idx = 0
    if emb:
        ksel_ref, bemb_ref = rest[idx], rest[idx + 1]
        idx += 2
    outs = rest[idx:]
    enew_ref, ecs_ref = outs[0], outs[1]
    acc_ref = outs[2]

    j = pl.program_id(0)
    nj = pl.num_programs(0)
    a = a_ref[...].astype(BF)
    b = b_ref[...].astype(BF)
    if emb:
        e = _mm(e_ref[...].astype(BF), ksel_ref[0]) + bemb_ref[...]
    else:
        e = e_ref[...]
    ebf = e.astype(BF)
    ub = u_ref[...] @ w1u_ref[...] + b1_ref[...]
    ubt = jnp.concatenate([ub, ub, ub, ub], axis=1)
    x1 = _lrelu(_mm(a, wa_ref[...]) + _mm(b, wb_ref[...])
                + _mm(ebf, we_ref[...]) + ubt)
    x2 = _lrelu(_mm(x1.astype(BF), w2_ref[...]) + b2_ref[...])
    en = _mm(x2.astype(BF), w3_ref[...]) + b3_ref[...] + e
    enew_ref[...] = en
    cs = jnp.sum(en, axis=0, keepdims=True)

    @pl.when(j == 0)
    def _():
        acc_ref[...] = cs

    @pl.when(j > 0)
    def _():
        acc_ref[...] = acc_ref[...] + cs

    @pl.when(j == nj - 1)
    def _():
        ecs_ref[...] = acc_ref[...] @ fold_ref[...]


def _edge_call(a, b, e_in, u, blk, emb_w, lastblk):
    phi = blk["phi_e"]
    w1full = phi[0]["w"]
    wa = _bd4(w1full[:32]).astype(BF)
    wb = _bd4(w1full[32:64]).astype(BF)
    we = _bd4(w1full[64:96]).astype(BF)
    w1u = w1full[96:]
    b1 = phi[0]["b"][None, :]
    w2 = _bd4(phi[1]["w"]).astype(BF)
    b2 = _tile4(phi[1]["b"])
    w3 = _bd4(phi[2]["w"]).astype(BF)
    b3 = _tile4(phi[2]["b"])
    fold = _fold4(32)

    C = EDGE_CHUNK
    nh = (EPK // C) // 2
    emb = emb_w is not None

    def row_spec(d, dt=None):
        return pl.BlockSpec((C, d), lambda j: (j, 0))

    def full_spec(shape):
        return pl.BlockSpec(shape, lambda j: (0,) * len(shape))

    if emb:
        e_spec = pl.BlockSpec(
            (C, 128), lambda j: (jnp.where(j < nh, j, j - nh), 0))
    else:
        e_spec = row_spec(128)

    in_specs = [row_spec(128), row_spec(128), e_spec,
                full_spec(u.shape),
                full_spec(wa.shape), full_spec(wb.shape), full_spec(we.shape),
                full_spec(w1u.shape), full_spec(b1.shape),
                full_spec(w2.shape), full_spec(b2.shape),
                full_spec(w3.shape), full_spec(b3.shape),
                full_spec(fold.shape)]
    args = [a, b, e_in, u, wa, wb, we, w1u, b1, w2, b2, w3, b3, fold]
    if emb:
        ksel, bemb = emb_w
        in_specs += [
            pl.BlockSpec((1, 128, 128),
                         lambda j: (jnp.where(j < nh, 0, 1), 0, 0)),
            full_spec(bemb.shape)]
        args += [ksel, bemb]

    out_specs = [row_spec(128), pl.BlockSpec((1, 32), lambda j: (0, 0))]
    out_shape = [jax.ShapeDtypeStruct((EPK, 128), jnp.float32),
                 jax.ShapeDtypeStruct((1, 32), jnp.float32)]

    return pl.pallas_call(
        functools.partial(_edge_body, emb, lastblk),
        grid=(EPK // C,),
        in_specs=in_specs,
        out_specs=out_specs,
        out_shape=out_shape,
        scratch_shapes=[pltpu.VMEM((1, 128), jnp.float32)],
        compiler_params=pltpu.CompilerParams(
            dimension_semantics=("arbitrary",)),
    )(*args)


def _node_body(firstblk, lastblk,
               h_ref, p0_ref, p1_ref, cnt_ref, u_ref, ecs_ref,
               wv1h_ref, wv1a_ref, wv1u_ref, bv1_ref,
               wv2_ref, bv2_ref, wv3_ref, bv3_ref, fold_ref,
               wu1h_ref, wu1e_ref, wu1u_ref, bu1_ref,
               wu2_ref, bu2_ref, wu3_ref, bu3_ref,
               *rest):
    idx = 0
    if firstblk:
        c1_ref = rest[idx]
        idx += 1
    if lastblk:
        wnl_ref, bnl_ref = rest[idx], rest[idx + 1]
        idx += 2
    outs = rest[idx:]
    hn_ref, un_ref = outs[0], outs[1]
    outs = outs[2:]
    if firstblk:
        cnt_out_ref = outs[0]
        outs = outs[1:]
    if lastblk:
        h16_ref = outs[0]

    h = h_ref[...]
    if firstblk:
        cnt = cnt_ref[...] + c1_ref[...]
        cnt_out_ref[...] = cnt
    else:
        cnt = cnt_ref[...]
    agg = (p0_ref[...] + p1_ref[...]) / jnp.maximum(cnt, 1.0)
    urow = u_ref[...]
    ub = urow @ wv1u_ref[...] + bv1_ref[...]
    ubt = jnp.concatenate([ub, ub, ub, ub], axis=1)
    x1 = _lrelu(_mm(h.astype(BF), wv1h_ref[...])
                + _mm(agg.astype(BF), wv1a_ref[...]) + ubt)
    x2 = _lrelu(_mm(x1.astype(BF), wv2_ref[...]) + bv2_ref[...])
    hn = _mm(x2.astype(BF), wv3_ref[...]) + bv3_ref[...] + h
    hn_ref[...] = hn

    hm = (jnp.sum(hn, axis=0, keepdims=True) @ fold_ref[...]) * (1.0 / N_NODES)
    em = ecs_ref[...] * (1.0 / N_EDGES)
    y1 = _lrelu(hm @ wu1h_ref[...] + em @ wu1e_ref[...]
                + urow @ wu1u_ref[...] + bu1_ref[...])
    y2 = _lrelu(y1 @ wu2_ref[...] + bu2_ref[...])
    un_ref[...] = y2 @ wu3_ref[...] + bu3_ref[...] + urow

    if lastblk:
        h16_ref[...] = hn @ wnl_ref[...] + bnl_ref[...]


def _node_call(h, p0, p1, cnt, c1, u, ecs, blk, params, firstblk, lastblk):
    phiv = blk["phi_v"]
    wv1full = phiv[0]["w"]
    wv1h = _bd4(wv1full[:32]).astype(BF)
    wv1a = _bd4(wv1full[32:64]).astype(BF)
    wv1u = wv1full[64:]
    bv1 = phiv[0]["b"][None, :]
    wv2 = _bd4(phiv[1]["w"]).astype(BF)
    bv2 = _tile4(phiv[1]["b"])
    wv3 = _bd4(phiv[2]["w"]).astype(BF)
    bv3 = _tile4(phiv[2]["b"])
    fold = _fold4(32)
    phiu = blk["phi_u"]
    wu1full = phiu[0]["w"]
    wu1h = wu1full[:32]
    wu1e = wu1full[32:64]
    wu1u = wu1full[64:]
    bu1 = phiu[0]["b"][None, :]
    wu2, bu2 = phiu[1]["w"], phiu[1]["b"][None, :]
    wu3, bu3 = phiu[2]["w"], phiu[2]["b"][None, :]

    args = [h, p0, p1, cnt, u, ecs,
            wv1h, wv1a, wv1u, bv1, wv2, bv2, wv3, bv3, fold,
            wu1h, wu1e, wu1u, bu1, wu2, bu2, wu3, bu3]
    out_shape = [jax.ShapeDtypeStruct((NPK, 128), jnp.float32),
                 jax.ShapeDtypeStruct((1, 32), jnp.float32)]
    if firstblk:
        args += [c1]
        out_shape.append(jax.ShapeDtypeStruct((NPK, 128), jnp.float32))
    if lastblk:
        wnl = _bd4(params["node_last"]["w"])
        bnl = _tile4(params["node_last"]["b"])
        args += [wnl, bnl]
        out_shape.append(jax.ShapeDtypeStruct((NPK, 64), jnp.float32))

    return pl.pallas_call(
        functools.partial(_node_body, firstblk, lastblk),
        out_shape=out_shape,
    )(*args)


def _s2s_body(nchunks, has_proj, D,
              *refs):
    if has_proj:
        (x_ref, welt_ref, wel_ref, bel_ref, bd_ref, bdt_ref, fold_ref,
         wih0_ref, whh0_ref, b0_ref, wih1_ref, whh1_ref, b1_ref,
         out_ref, st_ref) = refs
    else:
        (x_ref, bd_ref, bdt_ref, fold_ref,
         wih0_ref, whh0_ref, b0_ref, wih1_ref, whh1_ref, b1_ref,
         out_ref, st_ref) = refs
    i = pl.program_id(0)
    j = pl.program_id(1)
    L = 4 * D

    @pl.when(jnp.logical_and(i == 0, j == 0))
    def _():
        st_ref[...] = jnp.zeros((8, 128), jnp.float32)

    @pl.when(j == 0)
    def _():
        qs = st_ref[4:5, 0:32]
        h0 = st_ref[0:1, 0:16]
        c0 = st_ref[1:2, 0:16]
        h1 = st_ref[2:3, 0:16]
        c1 = st_ref[3:4, 0:16]
        g = qs @ wih0_ref[...] + h0 @ whh0_ref[...] + b0_ref[...]
        ig = _sigm(g[:, 0:16])
        fg = _sigm(g[:, 16:32])
        gg = _tanh(g[:, 32:48])
        og = _sigm(g[:, 48:64])
        c0n = fg * c0 + ig * gg
        h0n = og * _tanh(c0n)
        g2 = h0n @ wih1_ref[...] + h1 @ whh1_ref[...] + b1_ref[...]
        ig2 = _sigm(g2[:, 0:16])
        fg2 = _sigm(g2[:, 16:32])
        gg2 = _tanh(g2[:, 32:48])
        og2 = _sigm(g2[:, 48:64])
        c1n = fg2 * c1 + ig2 * gg2
        h1n = og2 * _tanh(c1n)
        st_ref[0:1, 0:16] = h0n
        st_ref[1:2, 0:16] = c0n
        st_ref[2:3, 0:16] = h1n
        st_ref[3:4, 0:16] = c1n
        st_ref[5:6, 0:16] = h1n
        st_ref[6:7, :] = jnp.zeros((1, 128), jnp.float32)
        st_ref[7:8, 0:1] = jnp.full((1, 1), -1e30, jnp.float32)
        st_ref[7:8, 1:2] = jnp.zeros((1, 1), jnp.float32)

    x = x_ref[...]
    q = st_ref[5:6, 0:16]
    if has_proj:
        qw = q @ welt_ref[...]
    else:
        qw = q
    qwl = jnp.concatenate([qw, qw, qw, qw], axis=1).astype(BF)
    xb = x if x.dtype == BF else x.astype(BF)
    sc4 = _mm(xb * qwl, bd_ref[...])
    m_old = st_ref[7:8, 0:1]
    s_old = st_ref[7:8, 1:2]
    cmax = jnp.max(sc4, axis=(0, 1), keepdims=True)
    m_new = jnp.maximum(m_old, cmax)
    scale = jnp.exp(m_old - m_new)
    w4 = jnp.exp(sc4 - m_new)
    ssum = jnp.sum(w4, axis=(0, 1), keepdims=True)
    wx = _mm(w4.astype(BF), bdt_ref[...])
    v = jnp.sum(wx * x.astype(jnp.float32), axis=0, keepdims=True)
    st_ref[6:7, 0:L] = st_ref[6:7, 0:L] * scale + v
    st_ref[7:8, 0:1] = m_new
    st_ref[7:8, 1:2] = s_old * scale + ssum

    @pl.when(j == nchunks - 1)
    def _():
        vd = (st_ref[6:7, 0:L] @ fold_ref[...]) / st_ref[7:8, 1:2]
        if has_proj:
            r = vd @ wel_ref[...] + bel_ref[...]
        else:
            r = vd
        qsn = jnp.concatenate([st_ref[5:6, 0:16], r], axis=1)
        st_ref[4:5, 0:32] = qsn

        @pl.when(i == 2)
        def _():
            out_ref[...] = qsn


def _s2s_call(x_packed, p, chunk, proj):
    R = x_packed.shape[0]
    L = x_packed.shape[1]
    D = L // 4
    nchunks = R // chunk
    ii = jnp.arange(L)
    bd = (ii[:, None] // D == jnp.arange(4)[None, :]).astype(BF)
    bdt = bd.T
    fold = (ii[:, None] % D == jnp.arange(D)[None, :]).astype(jnp.float32)
    wih0, whh0, b0 = p["wih0"], p["whh0"], p["b0"][None, :]
    wih1, whh1, b1 = p["wih1"], p["whh1"], p["b1"][None, :]

    def full_spec2(shape):
        return pl.BlockSpec(shape, lambda i, j: (0,) * len(shape))

    in_specs = [pl.BlockSpec((chunk, L), lambda i, j: (j, 0))]
    args = [x_packed]
    if proj is not None:
        wel, bel = proj
        welt = wel.T
        in_specs += [full_spec2(welt.shape), full_spec2(wel.shape),
                     full_spec2(bel.shape)]
        args += [welt, wel, bel]
    in_specs += [full_spec2(bd.shape), full_spec2(bdt.shape),
                 full_spec2(fold.shape),
                 full_spec2(wih0.shape), full_spec2(whh0.shape),
                 full_spec2(b0.shape),
                 full_spec2(wih1.shape), full_spec2(whh1.shape),
                 full_spec2(b1.shape)]
    args += [bd, bdt, fold, wih0, whh0, b0, wih1, whh1, b1]

    return pl.pallas_call(
        functools.partial(_s2s_body, nchunks, proj is not None, D),
        grid=(3, nchunks),
        in_specs=in_specs,
        out_specs=pl.BlockSpec((1, 32), lambda i, j: (0, 0)),
        out_shape=jax.ShapeDtypeStruct((1, 32), jnp.float32),
        scratch_shapes=[pltpu.VMEM((8, 128), jnp.float32)],
        compiler_params=pltpu.CompilerParams(
            dimension_semantics=("arbitrary", "arbitrary")),
    )(*args)


def _pred_body(hp_ref, ep_ref, w1h_ref, w1e_ref, b1_ref, w2_ref, b2_ref,
               out_ref):
    y1 = _lrelu(hp_ref[...] @ w1h_ref[...] + ep_ref[...] @ w1e_ref[...]
                + b1_ref[...])
    out_ref[...] = y1 @ w2_ref[...] + b2_ref[...]


def _pred_call(hp, ep, p):
    w1full = p[0]["w"]
    w1h = w1full[:32]
    w1e = w1full[32:]
    b1 = p[0]["b"][None, :]
    w2, b2 = p[1]["w"], p[1]["b"][None, :]
    return pl.pallas_call(
        _pred_body,
        out_shape=jax.ShapeDtypeStruct((1, 1), jnp.float32),
    )(hp, ep, w1h, w1e, b1, w2, b2)


def kernel(node_feats, edge_feats, graph_feats, edge_index, params):
    ei8 = edge_index.reshape(2, N_EDGES // 8, 8)
    src = jnp.concatenate([ei8[0, :, :4].reshape(-1),
                           ei8[0, :, 4:].reshape(-1)])
    dst = jnp.concatenate([ei8[1, :, :4].reshape(-1),
                           ei8[1, :, 4:].reshape(-1)])
    ef8 = jnp.reshape(edge_feats, (N_EDGES // 8, 128))

    h32, u = _emb_call(node_feats, graph_feats, params)

    wemb = params["edge_emb"]["w"]
    kemb = jnp.kron(jnp.eye(4, dtype=wemb.dtype), wemb)
    z64 = jnp.zeros((64, 128), jnp.float32)
    ksel = jnp.stack([jnp.concatenate([kemb, z64], axis=0),
                      jnp.concatenate([z64, kemb], axis=0)]).astype(BF)
    bemb = _tile4(params["edge_emb"]["b"])

    zeros_tile = jnp.zeros((N_NODES // SC_TILES, 32), jnp.float32)
    ones_k = jnp.ones((SCATTER_K, 32), jnp.float32)

    cnt_pk = None
    h16 = None
    e_in = ef8
    for bi, blk in enumerate(params["blocks"]):
        a32, b32 = _gather_call(h32, src, dst)
        a_pk = jnp.reshape(a32, (EPK, 128))
        b_pk = jnp.reshape(b32, (EPK, 128))
        firstblk = bi == 0
        lastblk = bi == 2
        h_pk = jnp.reshape(h32, (NPK, 128))
        emb_w = (ksel, bemb) if firstblk else None
        eouts = _edge_call(a_pk, b_pk, e_in, u, blk, emb_w, lastblk)
        enew_pk, ecs = eouts[0], eouts[1]
        souts = _scatter_call(jnp.reshape(enew_pk, (N_EDGES, 32)), dst,
                              zeros_tile, ones_k, firstblk)
        parts = jnp.reshape(souts[0], (2 * NPK, 128))
        p0 = parts[:NPK]
        p1 = parts[NPK:]
        if firstblk:
            cparts = jnp.reshape(souts[1], (2 * NPK, 128))
            c0 = cparts[:NPK]
            c1 = cparts[NPK:]
        else:
            c0, c1 = cnt_pk, None
        nouts = _node_call(h_pk, p0, p1, c0, c1, u, ecs, blk, params,
                           firstblk, lastblk)
        hn_pk, u = nouts[0], nouts[1]
        nouts = nouts[2:]
        if firstblk:
            cnt_pk = nouts[0]
            nouts = nouts[1:]
        if lastblk:
            h16 = nouts[0]
        h32 = jnp.reshape(hn_pk, (N_NODES, 32))
        e_in = enew_pk

    wel = params["edge_last"]["w"]
    bel = params["edge_last"]["b"][None, :]
    hp = _s2s_call(h16, params["s2s_node"], NPK, None)
    ep = _s2s_call(e_in, params["s2s_edge"], S2S_EDGE_CHUNK, (wel, bel))
    return _pred_call(hp, ep, params["pred"])


# speedup vs baseline: 1.7693x; 1.2442x over previous
import functools

import jax
import jax.numpy as jnp
from jax import lax
from jax.experimental import pallas as pl
from jax.experimental.pallas import tpu as pltpu
from jax.experimental.pallas import tpu_sc as plsc

N_NODES = 10000
N_EDGES = 320000
EPK = N_EDGES // 4
NPK = N_NODES // 4
EDGE_CHUNK = 4000
S2S_EDGE_CHUNK = 10000

SC_CORES = 2
SC_TILES = 16
SC_WORKERS = SC_CORES * SC_TILES
GATHER_K = 1000
SCATTER_K = 1000

BF = jnp.bfloat16


def _lrelu(x):
    return jnp.where(x > 0, x, 0.01 * x)


def _sigm(x):
    return 1.0 / (1.0 + jnp.exp(-x))


def _tanh(x):
    e2 = jnp.exp(-2.0 * x)
    return (1.0 - e2) / (1.0 + e2)


def _mm(x, w):
    return jnp.matmul(x, w, preferred_element_type=jnp.float32)


def _bd4(w):
    return jnp.kron(jnp.eye(4, dtype=w.dtype), w)


def _tile4(b):
    return jnp.tile(b, (4,))[None, :]


def _fold4(d):
    ii = jnp.arange(4 * d)
    return (ii[:, None] % d == jnp.arange(d)[None, :]).astype(jnp.float32)


def _sc_mesh():
    return plsc.VectorSubcoreMesh(core_axis_name="c", subcore_axis_name="s")


def _gather_call(h, src, dst):
    n2 = h.shape[1]
    epw = N_EDGES // SC_WORKERS
    k = GATHER_K
    niter = epw // k

    @functools.partial(
        pl.kernel,
        out_type=[jax.ShapeDtypeStruct((N_EDGES, n2), jnp.float32),
                  jax.ShapeDtypeStruct((N_EDGES, n2), jnp.float32)],
        mesh=_sc_mesh(),
        scratch_types=[pltpu.VMEM((k,), jnp.int32),
                       pltpu.VMEM((k,), jnp.int32),
                       pltpu.VMEM((k, n2), jnp.float32),
                       pltpu.VMEM((k, n2), jnp.float32),
                       pltpu.SemaphoreType.DMA,
                       pltpu.SemaphoreType.DMA],
        compiler_params=pltpu.CompilerParams(use_tc_tiling_on_sc=False),
    )
    def gk(h_hbm, src_hbm, dst_hbm, a_hbm, b_hbm,
           idx_s, idx_d, rows_a, rows_b, sem_a, sem_b):
        wid = lax.axis_index("s") * SC_CORES + lax.axis_index("c")
        base = wid * epw

        def body(t, carry):
            off = base + t * k
            pltpu.sync_copy(src_hbm.at[pl.ds(off, k)], idx_s)
            pltpu.sync_copy(dst_hbm.at[pl.ds(off, k)], idx_d)
            ca = pltpu.async_copy(h_hbm.at[idx_s], rows_a, sem_a)
            cb = pltpu.async_copy(h_hbm.at[idx_d], rows_b, sem_b)
            ca.wait()
            cb.wait()
            pltpu.sync_copy(rows_a, a_hbm.at[pl.ds(off, k)])
            pltpu.sync_copy(rows_b, b_hbm.at[pl.ds(off, k)])
            return carry

        lax.fori_loop(0, niter, body, 0)

    return gk(h, src, dst)


def _scatter_call(e_new, dst, zeros_tile, ones_k, with_counts):
    n2 = e_new.shape[1]
    epc = N_EDGES // SC_CORES
    epw = epc // SC_TILES
    k = SCATTER_K
    niter = epw // k
    stripe = N_NODES // SC_TILES

    out_type = [jax.ShapeDtypeStruct((SC_CORES * N_NODES, n2), jnp.float32)]
    scratch = [pltpu.VMEM((k,), jnp.int32),
               pltpu.VMEM((k, n2), jnp.float32),
               pltpu.VMEM_SHARED((N_NODES, n2), jnp.float32)]
    if with_counts:
        out_type.append(jax.ShapeDtypeStruct((SC_CORES * N_NODES, n2),
                                             jnp.float32))
        scratch.append(pltpu.VMEM_SHARED((N_NODES, n2), jnp.float32))
        scratch.append(pltpu.VMEM((k, n2), jnp.float32))

    @functools.partial(
        pl.kernel,
        out_type=out_type,
        mesh=_sc_mesh(),
        scratch_types=scratch,
        compiler_params=pltpu.CompilerParams(use_tc_tiling_on_sc=False),
    )
    def sk(e_hbm, dst_hbm, z_hbm, o_hbm, *rest):
        if with_counts:
            (p_hbm, c_hbm, idx_v, rows_v, shared, shared_cnt, ones_v) = rest
        else:
            (p_hbm, idx_v, rows_v, shared) = rest
        cid = lax.axis_index("c")
        sid = lax.axis_index("s")
        row0 = sid * stripe
        pltpu.sync_copy(z_hbm, shared.at[pl.ds(row0, stripe)])
        if with_counts:
            pltpu.sync_copy(z_hbm, shared_cnt.at[pl.ds(row0, stripe)])
            pltpu.sync_copy(o_hbm, ones_v)
        plsc.subcore_barrier()
        base = cid * epc + sid * epw

        def body(t, carry):
            off = base + t * k
            pltpu.sync_copy(dst_hbm.at[pl.ds(off, k)], idx_v)
            pltpu.sync_copy(e_hbm.at[pl.ds(off, k)], rows_v)
            pltpu.sync_copy(rows_v, shared.at[idx_v], add=True)
            if with_counts:
                pltpu.sync_copy(ones_v, shared_cnt.at[idx_v], add=True)
            return carry

        lax.fori_loop(0, niter, body, 0)
        plsc.subcore_barrier()
        obase = cid * N_NODES + row0
        pltpu.sync_copy(shared.at[pl.ds(row0, stripe)],
                        p_hbm.at[pl.ds(obase, stripe)])
        if with_counts:
            pltpu.sync_copy(shared_cnt.at[pl.ds(row0, stripe)],
                            c_hbm.at[pl.ds(obase, stripe)])

    return sk(e_new, dst, zeros_tile, ones_k)


def _emb_body(nf_ref, wn_ref, bn_ref, gf_ref, wg_ref, bg_ref,
              h_ref, u_ref):
    h_ref[...] = nf_ref[...] @ wn_ref[...] + bn_ref[...]
    u_ref[...] = gf_ref[...] @ wg_ref[...] + bg_ref[...]


def _emb_call(node_feats, graph_feats, params):
    wn = params["node_emb"]["w"]
    bn = params["node_emb"]["b"][None, :]
    wg = params["graph_emb"]["w"]
    bg = params["graph_emb"]["b"][None, :]
    n2 = wn.shape[1]
    return pl.pallas_call(
        _emb_body,
        out_shape=[
            jax.ShapeDtypeStruct((N_NODES, n2), jnp.float32),
            jax.ShapeDtypeStruct((1, n2), jnp.float32),
        ],
    )(node_feats, wn, bn, graph_feats, wg, bg)


def _edge_body(emb, lastblk,
               a_ref, b_ref, e_ref, u_ref,
               wa_ref, wb_ref, we_ref, w1u_ref, b1_ref,
               w2_ref, b2_ref, w3_ref, b3_ref, fold_ref,
               *rest):
    idx = 0
    if emb:
        ksel_ref, bemb_ref = rest[idx], rest[idx + 1]
        idx += 2
    outs = rest[idx:]
    enew_ref, ecs_ref = outs[0], outs[1]
    acc_ref = outs[2]

    j = pl.program_id(0)
    nj = pl.num_programs(0)
    a = a_ref[...].astype(BF)
    b = b_ref[...].astype(BF)
    if emb:
        e = _mm(e_ref[...].astype(BF), ksel_ref[...]) + bemb_ref[...]
    else:
        e = e_ref[...]
    ebf = e.astype(BF)
    ub = u_ref[...] @ w1u_ref[...] + b1_ref[...]
    ubt = jnp.concatenate([ub, ub, ub, ub], axis=1)
    x1 = _lrelu(_mm(a, wa_ref[...]) + _mm(b, wb_ref[...])
                + _mm(ebf, we_ref[...]) + ubt)
    x2 = _lrelu(_mm(x1.astype(BF), w2_ref[...]) + b2_ref[...])
    en = _mm(x2.astype(BF), w3_ref[...]) + b3_ref[...] + e
    enew_ref[...] = en
    cs = jnp.sum(en, axis=0, keepdims=True)

    @pl.when(j == 0)
    def _():
        acc_ref[...] = cs

    @pl.when(j > 0)
    def _():
        acc_ref[...] = acc_ref[...] + cs

    @pl.when(j == nj - 1)
    def _():
        ecs_ref[...] = acc_ref[...] @ fold_ref[...]


def _edge_call(a, b, e_in, u, blk, emb_w, lastblk):
    phi = blk["phi_e"]
    w1full = phi[0]["w"]
    wa = _bd4(w1full[:32]).astype(BF)
    wb = _bd4(w1full[32:64]).astype(BF)
    we = _bd4(w1full[64:96]).astype(BF)
    w1u = w1full[96:]
    b1 = phi[0]["b"][None, :]
    w2 = _bd4(phi[1]["w"]).astype(BF)
    b2 = _tile4(phi[1]["b"])
    w3 = _bd4(phi[2]["w"]).astype(BF)
    b3 = _tile4(phi[2]["b"])
    fold = _fold4(32)

    C = EDGE_CHUNK
    emb = emb_w is not None

    def row_spec(d):
        return pl.BlockSpec((C, d), lambda j: (j, 0))

    def full_spec(shape):
        return pl.BlockSpec(shape, lambda j: (0,) * len(shape))

    e_spec = row_spec(64) if emb else row_spec(128)

    in_specs = [row_spec(128), row_spec(128), e_spec,
                full_spec(u.shape),
                full_spec(wa.shape), full_spec(wb.shape), full_spec(we.shape),
                full_spec(w1u.shape), full_spec(b1.shape),
                full_spec(w2.shape), full_spec(b2.shape),
                full_spec(w3.shape), full_spec(b3.shape),
                full_spec(fold.shape)]
    args = [a, b, e_in, u, wa, wb, we, w1u, b1, w2, b2, w3, b3, fold]
    if emb:
        ksel, bemb = emb_w
        in_specs += [full_spec(ksel.shape), full_spec(bemb.shape)]
        args += [ksel, bemb]

    out_specs = [row_spec(128), pl.BlockSpec((1, 32), lambda j: (0, 0))]
    out_shape = [jax.ShapeDtypeStruct((EPK, 128), jnp.float32),
                 jax.ShapeDtypeStruct((1, 32), jnp.float32)]

    return pl.pallas_call(
        functools.partial(_edge_body, emb, lastblk),
        grid=(EPK // C,),
        in_specs=in_specs,
        out_specs=out_specs,
        out_shape=out_shape,
        scratch_shapes=[pltpu.VMEM((1, 128), jnp.float32)],
        compiler_params=pltpu.CompilerParams(
            dimension_semantics=("arbitrary",)),
    )(*args)


def _node_body(firstblk, lastblk,
               h_ref, p0_ref, p1_ref, cnt_ref, u_ref, ecs_ref,
               wv1h_ref, wv1a_ref, wv1u_ref, bv1_ref,
               wv2_ref, bv2_ref, wv3_ref, bv3_ref, fold_ref,
               wu1h_ref, wu1e_ref, wu1u_ref, bu1_ref,
               wu2_ref, bu2_ref, wu3_ref, bu3_ref,
               *rest):
    idx = 0
    if firstblk:
        c1_ref = rest[idx]
        idx += 1
    if lastblk:
        wnl_ref, bnl_ref = rest[idx], rest[idx + 1]
        idx += 2
    outs = rest[idx:]
    hn_ref, un_ref = outs[0], outs[1]
    outs = outs[2:]
    if firstblk:
        cnt_out_ref = outs[0]
        outs = outs[1:]
    if lastblk:
        h16_ref = outs[0]

    h = h_ref[...]
    if firstblk:
        cnt = cnt_ref[0] + c1_ref[0]
        cnt_out_ref[...] = cnt
    else:
        cnt = cnt_ref[...]
    agg = (p0_ref[0] + p1_ref[0]) / jnp.maximum(cnt, 1.0)
    urow = u_ref[...]
    ub = urow @ wv1u_ref[...] + bv1_ref[...]
    ubt = jnp.concatenate([ub, ub, ub, ub], axis=1)
    x1 = _lrelu(_mm(h.astype(BF), wv1h_ref[...])
                + _mm(agg.astype(BF), wv1a_ref[...]) + ubt)
    x2 = _lrelu(_mm(x1.astype(BF), wv2_ref[...]) + bv2_ref[...])
    hn = _mm(x2.astype(BF), wv3_ref[...]) + bv3_ref[...] + h
    hn_ref[...] = hn

    hm = (jnp.sum(hn, axis=0, keepdims=True) @ fold_ref[...]) * (1.0 / N_NODES)
    em = ecs_ref[...] * (1.0 / N_EDGES)
    y1 = _lrelu(hm @ wu1h_ref[...] + em @ wu1e_ref[...]
                + urow @ wu1u_ref[...] + bu1_ref[...])
    y2 = _lrelu(y1 @ wu2_ref[...] + bu2_ref[...])
    un_ref[...] = y2 @ wu3_ref[...] + bu3_ref[...] + urow

    if lastblk:
        h16_ref[...] = hn @ wnl_ref[...] + bnl_ref[...]


def _node_call(h, p0, p1, cnt, c1, u, ecs, blk, params, firstblk, lastblk):
    phiv = blk["phi_v"]
    wv1full = phiv[0]["w"]
    wv1h = _bd4(wv1full[:32]).astype(BF)
    wv1a = _bd4(wv1full[32:64]).astype(BF)
    wv1u = wv1full[64:]
    bv1 = phiv[0]["b"][None, :]
    wv2 = _bd4(phiv[1]["w"]).astype(BF)
    bv2 = _tile4(phiv[1]["b"])
    wv3 = _bd4(phiv[2]["w"]).astype(BF)
    bv3 = _tile4(phiv[2]["b"])
    fold = _fold4(32)
    phiu = blk["phi_u"]
    wu1full = phiu[0]["w"]
    wu1h = wu1full[:32]
    wu1e = wu1full[32:64]
    wu1u = wu1full[64:]
    bu1 = phiu[0]["b"][None, :]
    wu2, bu2 = phiu[1]["w"], phiu[1]["b"][None, :]
    wu3, bu3 = phiu[2]["w"], phiu[2]["b"][None, :]

    def full_spec(shape):
        return pl.BlockSpec(shape, lambda j: (0,) * len(shape))

    def half_spec(k):
        return pl.BlockSpec((1, NPK, 128), lambda j: (k, 0, 0))

    args = [h, p0, p1, cnt, u, ecs,
            wv1h, wv1a, wv1u, bv1, wv2, bv2, wv3, bv3, fold,
            wu1h, wu1e, wu1u, bu1, wu2, bu2, wu3, bu3]
    in_specs = [full_spec(h.shape), half_spec(0), half_spec(1)]
    in_specs += [half_spec(0) if firstblk else full_spec(cnt.shape)]
    in_specs += [full_spec(x.shape) for x in args[4:]]
    out_shape = [jax.ShapeDtypeStruct((NPK, 128), jnp.float32),
                 jax.ShapeDtypeStruct((1, 32), jnp.float32)]
    if firstblk:
        args += [c1]
        in_specs += [half_spec(1)]
        out_shape.append(jax.ShapeDtypeStruct((NPK, 128), jnp.float32))
    if lastblk:
        wnl = _bd4(params["node_last"]["w"])
        bnl = _tile4(params["node_last"]["b"])
        args += [wnl, bnl]
        in_specs += [full_spec(wnl.shape), full_spec(bnl.shape)]
        out_shape.append(jax.ShapeDtypeStruct((NPK, 64), jnp.float32))

    out_specs = [full_spec(s.shape) for s in out_shape]
    return pl.pallas_call(
        functools.partial(_node_body, firstblk, lastblk),
        grid=(1,),
        in_specs=in_specs,
        out_specs=out_specs,
        out_shape=out_shape,
    )(*args)


def _s2s_body(nchunks, has_proj,
              *refs):
    if has_proj:
        (x_ref, wproj_ref, bproj_ref, bd_ref, bdt_ref, f_ref, ft_ref,
         wih0_ref, whh0_ref, b0_ref, wih1_ref, whh1_ref, b1_ref,
         out_ref, st_ref) = refs
    else:
        (x_ref, bd_ref, bdt_ref, f_ref, ft_ref,
         wih0_ref, whh0_ref, b0_ref, wih1_ref, whh1_ref, b1_ref,
         out_ref, st_ref) = refs
    i = pl.program_id(0)
    j = pl.program_id(1)

    @pl.when(jnp.logical_and(i == 0, j == 0))
    def _():
        st_ref[...] = jnp.zeros((8, 128), jnp.float32)

    @pl.when(j == 0)
    def _():
        qs = st_ref[4:5, 0:32]
        h0 = st_ref[0:1, 0:16]
        c0 = st_ref[1:2, 0:16]
        h1 = st_ref[2:3, 0:16]
        c1 = st_ref[3:4, 0:16]
        g = qs @ wih0_ref[...] + h0 @ whh0_ref[...] + b0_ref[...]
        ig = _sigm(g[:, 0:16])
        fg = _sigm(g[:, 16:32])
        gg = _tanh(g[:, 32:48])
        og = _sigm(g[:, 48:64])
        c0n = fg * c0 + ig * gg
        h0n = og * _tanh(c0n)
        g2 = h0n @ wih1_ref[...] + h1 @ whh1_ref[...] + b1_ref[...]
        ig2 = _sigm(g2[:, 0:16])
        fg2 = _sigm(g2[:, 16:32])
        gg2 = _tanh(g2[:, 32:48])
        og2 = _sigm(g2[:, 48:64])
        c1n = fg2 * c1 + ig2 * gg2
        h1n = og2 * _tanh(c1n)
        st_ref[0:1, 0:16] = h0n
        st_ref[1:2, 0:16] = c0n
        st_ref[2:3, 0:16] = h1n
        st_ref[3:4, 0:16] = c1n
        st_ref[5:6, 0:16] = h1n
        st_ref[6:7, :] = jnp.zeros((1, 128), jnp.float32)
        st_ref[7:8, 0:1] = jnp.full((1, 1), -1e30, jnp.float32)
        st_ref[7:8, 1:2] = jnp.zeros((1, 1), jnp.float32)

    x = x_ref[...]
    if has_proj:
        x = x @ wproj_ref[...] + bproj_ref[...]
    q = st_ref[5:6, 0:16]
    q64 = q @ ft_ref[...]
    sc4 = (x * q64) @ bd_ref[...]
    m_old = st_ref[7:8, 0:1]
    s_old = st_ref[7:8, 1:2]
    cmax = jnp.max(sc4, axis=(0, 1), keepdims=True)
    m_new = jnp.maximum(m_old, cmax)
    scale = jnp.exp(m_old - m_new)
    w4 = jnp.exp(sc4 - m_new)
    ssum = jnp.sum(w4, axis=(0, 1), keepdims=True)
    wx = w4 @ bdt_ref[...]
    v = jnp.sum(wx * x, axis=0, keepdims=True)
    st_ref[6:7, 0:64] = st_ref[6:7, 0:64] * scale + v
    st_ref[7:8, 0:1] = m_new
    st_ref[7:8, 1:2] = s_old * scale + ssum

    @pl.when(j == nchunks - 1)
    def _():
        r = (st_ref[6:7, 0:64] @ f_ref[...]) / st_ref[7:8, 1:2]
        qsn = jnp.concatenate([st_ref[5:6, 0:16], r], axis=1)
        st_ref[4:5, 0:32] = qsn

        @pl.when(i == 2)
        def _():
            out_ref[...] = qsn


def _s2s_call(x_packed, p, chunk, proj):
    R = x_packed.shape[0]
    L = x_packed.shape[1]
    nchunks = R // chunk
    d = 16
    ii = jnp.arange(64)
    bd = (ii[:, None] // d == jnp.arange(4)[None, :]).astype(jnp.float32)
    f = (ii[:, None] % d == jnp.arange(d)[None, :]).astype(jnp.float32)
    bdt = bd.T
    ft = f.T
    wih0, whh0, b0 = p["wih0"], p["whh0"], p["b0"][None, :]
    wih1, whh1, b1 = p["wih1"], p["whh1"], p["b1"][None, :]

    def full_spec2(shape):
        return pl.BlockSpec(shape, lambda i, j: (0,) * len(shape))

    in_specs = [pl.BlockSpec((chunk, L), lambda i, j: (j, 0))]
    args = [x_packed]
    if proj is not None:
        wproj, bproj = proj
        in_specs += [full_spec2(wproj.shape), full_spec2(bproj.shape)]
        args += [wproj, bproj]
    in_specs += [full_spec2(bd.shape), full_spec2(bdt.shape),
                 full_spec2(f.shape), full_spec2(ft.shape),
                 full_spec2(wih0.shape), full_spec2(whh0.shape),
                 full_spec2(b0.shape),
                 full_spec2(wih1.shape), full_spec2(whh1.shape),
                 full_spec2(b1.shape)]
    args += [bd, bdt, f, ft, wih0, whh0, b0, wih1, whh1, b1]

    return pl.pallas_call(
        functools.partial(_s2s_body, nchunks, proj is not None),
        grid=(3, nchunks),
        in_specs=in_specs,
        out_specs=pl.BlockSpec((1, 32), lambda i, j: (0, 0)),
        out_shape=jax.ShapeDtypeStruct((1, 32), jnp.float32),
        scratch_shapes=[pltpu.VMEM((8, 128), jnp.float32)],
        compiler_params=pltpu.CompilerParams(
            dimension_semantics=("arbitrary", "arbitrary")),
    )(*args)


def _pred_body(hp_ref, ep_ref, w1h_ref, w1e_ref, b1_ref, w2_ref, b2_ref,
               out_ref):
    y1 = _lrelu(hp_ref[...] @ w1h_ref[...] + ep_ref[...] @ w1e_ref[...]
                + b1_ref[...])
    out_ref[...] = y1 @ w2_ref[...] + b2_ref[...]


def _pred_call(hp, ep, p):
    w1full = p[0]["w"]
    w1h = w1full[:32]
    w1e = w1full[32:]
    b1 = p[0]["b"][None, :]
    w2, b2 = p[1]["w"], p[1]["b"][None, :]
    return pl.pallas_call(
        _pred_body,
        out_shape=jax.ShapeDtypeStruct((1, 1), jnp.float32),
    )(hp, ep, w1h, w1e, b1, w2, b2)


def kernel(node_feats, edge_feats, graph_feats, edge_index, params):
    src = edge_index[0]
    dst = edge_index[1]
    ef4 = jnp.reshape(edge_feats, (EPK, 64))

    h32, u = _emb_call(node_feats, graph_feats, params)

    wemb = params["edge_emb"]["w"]
    ksel = jnp.kron(jnp.eye(4, dtype=wemb.dtype), wemb).astype(BF)
    bemb = _tile4(params["edge_emb"]["b"])

    zeros_tile = jnp.zeros((N_NODES // SC_TILES, 32), jnp.float32)
    ones_k = jnp.ones((SCATTER_K, 32), jnp.float32)

    cnt_pk = None
    h16 = None
    e_in = ef4
    for bi, blk in enumerate(params["blocks"]):
        a32, b32 = _gather_call(h32, src, dst)
        a_pk = jnp.reshape(a32, (EPK, 128))
        b_pk = jnp.reshape(b32, (EPK, 128))
        firstblk = bi == 0
        lastblk = bi == 2
        h_pk = jnp.reshape(h32, (NPK, 128))
        emb_w = (ksel, bemb) if firstblk else None
        eouts = _edge_call(a_pk, b_pk, e_in, u, blk, emb_w, lastblk)
        enew_pk, ecs = eouts[0], eouts[1]
        souts = _scatter_call(jnp.reshape(enew_pk, (N_EDGES, 32)), dst,
                              zeros_tile, ones_k, firstblk)
        parts = jnp.reshape(souts[0], (2, NPK, 128))
        if firstblk:
            cparts = jnp.reshape(souts[1], (2, NPK, 128))
            c0, c1 = cparts, cparts
        else:
            c0, c1 = cnt_pk, None
        nouts = _node_call(h_pk, parts, parts, c0, c1, u, ecs, blk, params,
                           firstblk, lastblk)
        hn_pk, u = nouts[0], nouts[1]
        nouts = nouts[2:]
        if firstblk:
            cnt_pk = nouts[0]
            nouts = nouts[1:]
        if lastblk:
            h16 = nouts[0]
        h32 = jnp.reshape(hn_pk, (N_NODES, 32))
        e_in = enew_pk

    wel = _bd4(params["edge_last"]["w"])
    bel = _tile4(params["edge_last"]["b"])
    hp = _s2s_call(h16, params["s2s_node"], NPK, None)
    ep = _s2s_call(e_in, params["s2s_edge"], S2S_EDGE_CHUNK, (wel, bel))
    return _pred_call(hp, ep, params["pred"])


# speedup vs baseline: 1.7818x; 1.0071x over previous
import functools

import jax
import jax.numpy as jnp
from jax import lax
from jax.experimental import pallas as pl
from jax.experimental.pallas import tpu as pltpu
from jax.experimental.pallas import tpu_sc as plsc

N_NODES = 10000
N_EDGES = 320000
EPK = N_EDGES // 4
NPK = N_NODES // 4
EDGE_CHUNK = 4000
S2S_EDGE_CHUNK = 10000

SC_CORES = 2
SC_TILES = 16
SC_WORKERS = SC_CORES * SC_TILES
GATHER_K = 1000
SCATTER_K = 1000

BF = jnp.bfloat16


def _lrelu(x):
    return jnp.where(x > 0, x, 0.01 * x)


def _sigm(x):
    return 1.0 / (1.0 + jnp.exp(-x))


def _tanh(x):
    e2 = jnp.exp(-2.0 * x)
    return (1.0 - e2) / (1.0 + e2)


def _mm(x, w):
    return jnp.matmul(x, w, preferred_element_type=jnp.float32)


def _bd4(w):
    return jnp.kron(jnp.eye(4, dtype=w.dtype), w)


def _tile4(b):
    return jnp.tile(b, (4,))[None, :]


def _fold4(d):
    ii = jnp.arange(4 * d)
    return (ii[:, None] % d == jnp.arange(d)[None, :]).astype(jnp.float32)


def _sc_mesh():
    return plsc.VectorSubcoreMesh(core_axis_name="c", subcore_axis_name="s")


def _gather_call(h, src, dst):
    n2 = h.shape[1]
    epw = N_EDGES // SC_WORKERS
    k = GATHER_K
    niter = epw // k

    @functools.partial(
        pl.kernel,
        out_type=[jax.ShapeDtypeStruct((N_EDGES, n2), jnp.float32),
                  jax.ShapeDtypeStruct((N_EDGES, n2), jnp.float32)],
        mesh=_sc_mesh(),
        scratch_types=[pltpu.VMEM((k,), jnp.int32),
                       pltpu.VMEM((k,), jnp.int32),
                       pltpu.VMEM((k, n2), jnp.float32),
                       pltpu.VMEM((k, n2), jnp.float32),
                       pltpu.SemaphoreType.DMA,
                       pltpu.SemaphoreType.DMA],
        compiler_params=pltpu.CompilerParams(use_tc_tiling_on_sc=False),
    )
    def gk(h_hbm, src_hbm, dst_hbm, a_hbm, b_hbm,
           idx_s, idx_d, rows_a, rows_b, sem_a, sem_b):
        wid = lax.axis_index("s") * SC_CORES + lax.axis_index("c")
        base = wid * epw

        def body(t, carry):
            off = base + t * k
            pltpu.sync_copy(src_hbm.at[pl.ds(off, k)], idx_s)
            pltpu.sync_copy(dst_hbm.at[pl.ds(off, k)], idx_d)
            ca = pltpu.async_copy(h_hbm.at[idx_s], rows_a, sem_a)
            cb = pltpu.async_copy(h_hbm.at[idx_d], rows_b, sem_b)
            ca.wait()
            cb.wait()
            pltpu.sync_copy(rows_a, a_hbm.at[pl.ds(off, k)])
            pltpu.sync_copy(rows_b, b_hbm.at[pl.ds(off, k)])
            return carry

        lax.fori_loop(0, niter, body, 0)

    return gk(h, src, dst)


def _scatter_call(e_new, dst, zeros_tile, ones_k, with_counts):
    n2 = e_new.shape[1]
    epc = N_EDGES // SC_CORES
    epw = epc // SC_TILES
    k = SCATTER_K
    niter = epw // k
    stripe = N_NODES // SC_TILES

    out_type = [jax.ShapeDtypeStruct((SC_CORES * N_NODES, n2), jnp.float32)]
    scratch = [pltpu.VMEM((k,), jnp.int32),
               pltpu.VMEM((k, n2), jnp.float32),
               pltpu.VMEM_SHARED((N_NODES, n2), jnp.float32)]
    if with_counts:
        out_type.append(jax.ShapeDtypeStruct((SC_CORES * N_NODES, n2),
                                             jnp.float32))
        scratch.append(pltpu.VMEM_SHARED((N_NODES, n2), jnp.float32))
        scratch.append(pltpu.VMEM((k, n2), jnp.float32))

    @functools.partial(
        pl.kernel,
        out_type=out_type,
        mesh=_sc_mesh(),
        scratch_types=scratch,
        compiler_params=pltpu.CompilerParams(use_tc_tiling_on_sc=False),
    )
    def sk(e_hbm, dst_hbm, z_hbm, o_hbm, *rest):
        if with_counts:
            (p_hbm, c_hbm, idx_v, rows_v, shared, shared_cnt, ones_v) = rest
        else:
            (p_hbm, idx_v, rows_v, shared) = rest
        cid = lax.axis_index("c")
        sid = lax.axis_index("s")
        row0 = sid * stripe
        pltpu.sync_copy(z_hbm, shared.at[pl.ds(row0, stripe)])
        if with_counts:
            pltpu.sync_copy(z_hbm, shared_cnt.at[pl.ds(row0, stripe)])
            pltpu.sync_copy(o_hbm, ones_v)
        plsc.subcore_barrier()
        base = cid * epc + sid * epw

        def body(t, carry):
            off = base + t * k
            pltpu.sync_copy(dst_hbm.at[pl.ds(off, k)], idx_v)
            pltpu.sync_copy(e_hbm.at[pl.ds(off, k)], rows_v)
            pltpu.sync_copy(rows_v, shared.at[idx_v], add=True)
            if with_counts:
                pltpu.sync_copy(ones_v, shared_cnt.at[idx_v], add=True)
            return carry

        lax.fori_loop(0, niter, body, 0)
        plsc.subcore_barrier()
        obase = cid * N_NODES + row0
        pltpu.sync_copy(shared.at[pl.ds(row0, stripe)],
                        p_hbm.at[pl.ds(obase, stripe)])
        if with_counts:
            pltpu.sync_copy(shared_cnt.at[pl.ds(row0, stripe)],
                            c_hbm.at[pl.ds(obase, stripe)])

    return sk(e_new, dst, zeros_tile, ones_k)


def _emb_body(nf_ref, wn_ref, bn_ref, gf_ref, wg_ref, bg_ref,
              h_ref, u_ref):
    h_ref[...] = nf_ref[...] @ wn_ref[...] + bn_ref[...]
    u_ref[...] = gf_ref[...] @ wg_ref[...] + bg_ref[...]


def _emb_call(node_feats, graph_feats, params):
    wn = params["node_emb"]["w"]
    bn = params["node_emb"]["b"][None, :]
    wg = params["graph_emb"]["w"]
    bg = params["graph_emb"]["b"][None, :]
    n2 = wn.shape[1]
    return pl.pallas_call(
        _emb_body,
        out_shape=[
            jax.ShapeDtypeStruct((N_NODES, n2), jnp.float32),
            jax.ShapeDtypeStruct((1, n2), jnp.float32),
        ],
    )(node_feats, wn, bn, graph_feats, wg, bg)


def _edge_body(emb, lastblk,
               a_ref, b_ref, e_ref, u_ref,
               wa_ref, wb_ref, we_ref, w1u_ref, b1_ref,
               w2_ref, b2_ref, w3_ref, b3_ref, fold_ref,
               *rest):
    idx = 0
    if emb:
        ksel_ref, bemb_ref = rest[idx], rest[idx + 1]
        idx += 2
    outs = rest[idx:]
    enew_ref, ecs_ref = outs[0], outs[1]
    acc_ref = outs[2]

    j = pl.program_id(0)
    nj = pl.num_programs(0)
    a = a_ref[...]
    b = b_ref[...]
    if emb:
        e = _mm(e_ref[...], ksel_ref[...]) + bemb_ref[...]
    else:
        e = e_ref[...]
    ebf = e
    ub = u_ref[...] @ w1u_ref[...] + b1_ref[...]
    ubt = jnp.concatenate([ub, ub, ub, ub], axis=1)
    x1 = _lrelu(_mm(a, wa_ref[...]) + _mm(b, wb_ref[...])
                + _mm(ebf, we_ref[...]) + ubt)
    x2 = _lrelu(_mm(x1, w2_ref[...]) + b2_ref[...])
    en = _mm(x2, w3_ref[...]) + b3_ref[...] + e
    enew_ref[...] = en
    cs = jnp.sum(en, axis=0, keepdims=True)

    @pl.when(j == 0)
    def _():
        acc_ref[...] = cs

    @pl.when(j > 0)
    def _():
        acc_ref[...] = acc_ref[...] + cs

    @pl.when(j == nj - 1)
    def _():
        ecs_ref[...] = acc_ref[...] @ fold_ref[...]


def _edge_call(a, b, e_in, u, blk, emb_w, lastblk):
    phi = blk["phi_e"]
    w1full = phi[0]["w"]
    wa = _bd4(w1full[:32])
    wb = _bd4(w1full[32:64])
    we = _bd4(w1full[64:96])
    w1u = w1full[96:]
    b1 = phi[0]["b"][None, :]
    w2 = _bd4(phi[1]["w"])
    b2 = _tile4(phi[1]["b"])
    w3 = _bd4(phi[2]["w"])
    b3 = _tile4(phi[2]["b"])
    fold = _fold4(32)

    C = EDGE_CHUNK
    emb = emb_w is not None

    def row_spec(d):
        return pl.BlockSpec((C, d), lambda j: (j, 0))

    def full_spec(shape):
        return pl.BlockSpec(shape, lambda j: (0,) * len(shape))

    e_spec = row_spec(64) if emb else row_spec(128)

    in_specs = [row_spec(128), row_spec(128), e_spec,
                full_spec(u.shape),
                full_spec(wa.shape), full_spec(wb.shape), full_spec(we.shape),
                full_spec(w1u.shape), full_spec(b1.shape),
                full_spec(w2.shape), full_spec(b2.shape),
                full_spec(w3.shape), full_spec(b3.shape),
                full_spec(fold.shape)]
    args = [a, b, e_in, u, wa, wb, we, w1u, b1, w2, b2, w3, b3, fold]
    if emb:
        ksel, bemb = emb_w
        in_specs += [full_spec(ksel.shape), full_spec(bemb.shape)]
        args += [ksel, bemb]

    out_specs = [row_spec(128), pl.BlockSpec((1, 32), lambda j: (0, 0))]
    out_shape = [jax.ShapeDtypeStruct((EPK, 128), jnp.float32),
                 jax.ShapeDtypeStruct((1, 32), jnp.float32)]

    return pl.pallas_call(
        functools.partial(_edge_body, emb, lastblk),
        grid=(EPK // C,),
        in_specs=in_specs,
        out_specs=out_specs,
        out_shape=out_shape,
        scratch_shapes=[pltpu.VMEM((1, 128), jnp.float32)],
        compiler_params=pltpu.CompilerParams(
            dimension_semantics=("arbitrary",)),
    )(*args)


def _node_body(firstblk, lastblk,
               h_ref, p0_ref, p1_ref, cnt_ref, u_ref, ecs_ref,
               wv1h_ref, wv1a_ref, wv1u_ref, bv1_ref,
               wv2_ref, bv2_ref, wv3_ref, bv3_ref, fold_ref,
               wu1h_ref, wu1e_ref, wu1u_ref, bu1_ref,
               wu2_ref, bu2_ref, wu3_ref, bu3_ref,
               *rest):
    idx = 0
    if firstblk:
        c1_ref = rest[idx]
        idx += 1
    if lastblk:
        wnl_ref, bnl_ref = rest[idx], rest[idx + 1]
        idx += 2
    outs = rest[idx:]
    hn_ref, un_ref = outs[0], outs[1]
    outs = outs[2:]
    if firstblk:
        cnt_out_ref = outs[0]
        outs = outs[1:]
    if lastblk:
        h16_ref = outs[0]

    h = h_ref[...]
    if firstblk:
        cnt = cnt_ref[0] + c1_ref[0]
        cnt_out_ref[...] = cnt
    else:
        cnt = cnt_ref[...]
    agg = (p0_ref[0] + p1_ref[0]) / jnp.maximum(cnt, 1.0)
    urow = u_ref[...]
    ub = urow @ wv1u_ref[...] + bv1_ref[...]
    ubt = jnp.concatenate([ub, ub, ub, ub], axis=1)
    x1 = _lrelu(_mm(h, wv1h_ref[...]) + _mm(agg, wv1a_ref[...]) + ubt)
    x2 = _lrelu(_mm(x1, wv2_ref[...]) + bv2_ref[...])
    hn = _mm(x2, wv3_ref[...]) + bv3_ref[...] + h
    hn_ref[...] = hn

    hm = (jnp.sum(hn, axis=0, keepdims=True) @ fold_ref[...]) * (1.0 / N_NODES)
    em = ecs_ref[...] * (1.0 / N_EDGES)
    y1 = _lrelu(hm @ wu1h_ref[...] + em @ wu1e_ref[...]
                + urow @ wu1u_ref[...] + bu1_ref[...])
    y2 = _lrelu(y1 @ wu2_ref[...] + bu2_ref[...])
    un_ref[...] = y2 @ wu3_ref[...] + bu3_ref[...] + urow

    if lastblk:
        h16_ref[...] = hn @ wnl_ref[...] + bnl_ref[...]


def _node_call(h, p0, p1, cnt, c1, u, ecs, blk, params, firstblk, lastblk):
    phiv = blk["phi_v"]
    wv1full = phiv[0]["w"]
    wv1h = _bd4(wv1full[:32])
    wv1a = _bd4(wv1full[32:64])
    wv1u = wv1full[64:]
    bv1 = phiv[0]["b"][None, :]
    wv2 = _bd4(phiv[1]["w"])
    bv2 = _tile4(phiv[1]["b"])
    wv3 = _bd4(phiv[2]["w"])
    bv3 = _tile4(phiv[2]["b"])
    fold = _fold4(32)
    phiu = blk["phi_u"]
    wu1full = phiu[0]["w"]
    wu1h = wu1full[:32]
    wu1e = wu1full[32:64]
    wu1u = wu1full[64:]
    bu1 = phiu[0]["b"][None, :]
    wu2, bu2 = phiu[1]["w"], phiu[1]["b"][None, :]
    wu3, bu3 = phiu[2]["w"], phiu[2]["b"][None, :]

    def full_spec(shape):
        return pl.BlockSpec(shape, lambda j: (0,) * len(shape))

    def half_spec(k):
        return pl.BlockSpec((1, NPK, 128), lambda j: (k, 0, 0))

    args = [h, p0, p1, cnt, u, ecs,
            wv1h, wv1a, wv1u, bv1, wv2, bv2, wv3, bv3, fold,
            wu1h, wu1e, wu1u, bu1, wu2, bu2, wu3, bu3]
    in_specs = [full_spec(h.shape), half_spec(0), half_spec(1)]
    in_specs += [half_spec(0) if firstblk else full_spec(cnt.shape)]
    in_specs += [full_spec(x.shape) for x in args[4:]]
    out_shape = [jax.ShapeDtypeStruct((NPK, 128), jnp.float32),
                 jax.ShapeDtypeStruct((1, 32), jnp.float32)]
    if firstblk:
        args += [c1]
        in_specs += [half_spec(1)]
        out_shape.append(jax.ShapeDtypeStruct((NPK, 128), jnp.float32))
    if lastblk:
        wnl = _bd4(params["node_last"]["w"])
        bnl = _tile4(params["node_last"]["b"])
        args += [wnl, bnl]
        in_specs += [full_spec(wnl.shape), full_spec(bnl.shape)]
        out_shape.append(jax.ShapeDtypeStruct((NPK, 64), jnp.float32))

    out_specs = [full_spec(s.shape) for s in out_shape]
    return pl.pallas_call(
        functools.partial(_node_body, firstblk, lastblk),
        grid=(1,),
        in_specs=in_specs,
        out_specs=out_specs,
        out_shape=out_shape,
    )(*args)


def _s2s_body(nchunks, has_proj,
              *refs):
    if has_proj:
        (x_ref, wproj_ref, bproj_ref, bd_ref, bdt_ref, f_ref, ft_ref,
         wih0_ref, whh0_ref, b0_ref, wih1_ref, whh1_ref, b1_ref,
         out_ref, st_ref) = refs
    else:
        (x_ref, bd_ref, bdt_ref, f_ref, ft_ref,
         wih0_ref, whh0_ref, b0_ref, wih1_ref, whh1_ref, b1_ref,
         out_ref, st_ref) = refs
    i = pl.program_id(0)
    j = pl.program_id(1)

    @pl.when(jnp.logical_and(i == 0, j == 0))
    def _():
        st_ref[...] = jnp.zeros((8, 128), jnp.float32)

    @pl.when(j == 0)
    def _():
        qs = st_ref[4:5, 0:32]
        h0 = st_ref[0:1, 0:16]
        c0 = st_ref[1:2, 0:16]
        h1 = st_ref[2:3, 0:16]
        c1 = st_ref[3:4, 0:16]
        g = qs @ wih0_ref[...] + h0 @ whh0_ref[...] + b0_ref[...]
        ig = _sigm(g[:, 0:16])
        fg = _sigm(g[:, 16:32])
        gg = _tanh(g[:, 32:48])
        og = _sigm(g[:, 48:64])
        c0n = fg * c0 + ig * gg
        h0n = og * _tanh(c0n)
        g2 = h0n @ wih1_ref[...] + h1 @ whh1_ref[...] + b1_ref[...]
        ig2 = _sigm(g2[:, 0:16])
        fg2 = _sigm(g2[:, 16:32])
        gg2 = _tanh(g2[:, 32:48])
        og2 = _sigm(g2[:, 48:64])
        c1n = fg2 * c1 + ig2 * gg2
        h1n = og2 * _tanh(c1n)
        st_ref[0:1, 0:16] = h0n
        st_ref[1:2, 0:16] = c0n
        st_ref[2:3, 0:16] = h1n
        st_ref[3:4, 0:16] = c1n
        st_ref[5:6, 0:16] = h1n
        st_ref[6:7, :] = jnp.zeros((1, 128), jnp.float32)
        st_ref[7:8, 0:1] = jnp.full((1, 1), -1e30, jnp.float32)
        st_ref[7:8, 1:2] = jnp.zeros((1, 1), jnp.float32)

    x = x_ref[...]
    if has_proj:
        x = x @ wproj_ref[...] + bproj_ref[...]
    q = st_ref[5:6, 0:16]
    q64 = q @ ft_ref[...]
    sc4 = (x * q64) @ bd_ref[...]
    m_old = st_ref[7:8, 0:1]
    s_old = st_ref[7:8, 1:2]
    cmax = jnp.max(sc4, axis=(0, 1), keepdims=True)
    m_new = jnp.maximum(m_old, cmax)
    scale = jnp.exp(m_old - m_new)
    w4 = jnp.exp(sc4 - m_new)
    ssum = jnp.sum(w4, axis=(0, 1), keepdims=True)
    wx = w4 @ bdt_ref[...]
    v = jnp.sum(wx * x, axis=0, keepdims=True)
    st_ref[6:7, 0:64] = st_ref[6:7, 0:64] * scale + v
    st_ref[7:8, 0:1] = m_new
    st_ref[7:8, 1:2] = s_old * scale + ssum

    @pl.when(j == nchunks - 1)
    def _():
        r = (st_ref[6:7, 0:64] @ f_ref[...]) / st_ref[7:8, 1:2]
        qsn = jnp.concatenate([st_ref[5:6, 0:16], r], axis=1)
        st_ref[4:5, 0:32] = qsn

        @pl.when(i == 2)
        def _():
            out_ref[...] = qsn


def _s2s_call(x_packed, p, chunk, proj):
    R = x_packed.shape[0]
    L = x_packed.shape[1]
    nchunks = R // chunk
    d = 16
    ii = jnp.arange(64)
    bd = (ii[:, None] // d == jnp.arange(4)[None, :]).astype(jnp.float32)
    f = (ii[:, None] % d == jnp.arange(d)[None, :]).astype(jnp.float32)
    bdt = bd.T
    ft = f.T
    wih0, whh0, b0 = p["wih0"], p["whh0"], p["b0"][None, :]
    wih1, whh1, b1 = p["wih1"], p["whh1"], p["b1"][None, :]

    def full_spec2(shape):
        return pl.BlockSpec(shape, lambda i, j: (0,) * len(shape))

    in_specs = [pl.BlockSpec((chunk, L), lambda i, j: (j, 0))]
    args = [x_packed]
    if proj is not None:
        wproj, bproj = proj
        in_specs += [full_spec2(wproj.shape), full_spec2(bproj.shape)]
        args += [wproj, bproj]
    in_specs += [full_spec2(bd.shape), full_spec2(bdt.shape),
                 full_spec2(f.shape), full_spec2(ft.shape),
                 full_spec2(wih0.shape), full_spec2(whh0.shape),
                 full_spec2(b0.shape),
                 full_spec2(wih1.shape), full_spec2(whh1.shape),
                 full_spec2(b1.shape)]
    args += [bd, bdt, f, ft, wih0, whh0, b0, wih1, whh1, b1]

    return pl.pallas_call(
        functools.partial(_s2s_body, nchunks, proj is not None),
        grid=(3, nchunks),
        in_specs=in_specs,
        out_specs=pl.BlockSpec((1, 32), lambda i, j: (0, 0)),
        out_shape=jax.ShapeDtypeStruct((1, 32), jnp.float32),
        scratch_shapes=[pltpu.VMEM((8, 128), jnp.float32)],
        compiler_params=pltpu.CompilerParams(
            dimension_semantics=("arbitrary", "arbitrary")),
    )(*args)


def _pred_body(hp_ref, ep_ref, w1h_ref, w1e_ref, b1_ref, w2_ref, b2_ref,
               out_ref):
    y1 = _lrelu(hp_ref[...] @ w1h_ref[...] + ep_ref[...] @ w1e_ref[...]
                + b1_ref[...])
    out_ref[...] = y1 @ w2_ref[...] + b2_ref[...]


def _pred_call(hp, ep, p):
    w1full = p[0]["w"]
    w1h = w1full[:32]
    w1e = w1full[32:]
    b1 = p[0]["b"][None, :]
    w2, b2 = p[1]["w"], p[1]["b"][None, :]
    return pl.pallas_call(
        _pred_body,
        out_shape=jax.ShapeDtypeStruct((1, 1), jnp.float32),
    )(hp, ep, w1h, w1e, b1, w2, b2)


def kernel(node_feats, edge_feats, graph_feats, edge_index, params):
    src = edge_index[0]
    dst = edge_index[1]
    ef4 = jnp.reshape(edge_feats, (EPK, 64))

    h32, u = _emb_call(node_feats, graph_feats, params)

    wemb = params["edge_emb"]["w"]
    ksel = jnp.kron(jnp.eye(4, dtype=wemb.dtype), wemb)
    bemb = _tile4(params["edge_emb"]["b"])

    zeros_tile = jnp.zeros((N_NODES // SC_TILES, 32), jnp.float32)
    ones_k = jnp.ones((SCATTER_K, 32), jnp.float32)

    cnt_pk = None
    h16 = None
    e_in = ef4
    for bi, blk in enumerate(params["blocks"]):
        a32, b32 = _gather_call(h32, src, dst)
        a_pk = jnp.reshape(a32, (EPK, 128))
        b_pk = jnp.reshape(b32, (EPK, 128))
        firstblk = bi == 0
        lastblk = bi == 2
        h_pk = jnp.reshape(h32, (NPK, 128))
        emb_w = (ksel, bemb) if firstblk else None
        eouts = _edge_call(a_pk, b_pk, e_in, u, blk, emb_w, lastblk)
        enew_pk, ecs = eouts[0], eouts[1]
        souts = _scatter_call(jnp.reshape(enew_pk, (N_EDGES, 32)), dst,
                              zeros_tile, ones_k, firstblk)
        parts = jnp.reshape(souts[0], (2, NPK, 128))
        if firstblk:
            cparts = jnp.reshape(souts[1], (2, NPK, 128))
            c0, c1 = cparts, cparts
        else:
            c0, c1 = cnt_pk, None
        nouts = _node_call(h_pk, parts, parts, c0, c1, u, ecs, blk, params,
                           firstblk, lastblk)
        hn_pk, u = nouts[0], nouts[1]
        nouts = nouts[2:]
        if firstblk:
            cnt_pk = nouts[0]
            nouts = nouts[1:]
        if lastblk:
            h16 = nouts[0]
        h32 = jnp.reshape(hn_pk, (N_NODES, 32))
        e_in = enew_pk

    wel = _bd4(params["edge_last"]["w"])
    bel = _tile4(params["edge_last"]["b"])
    hp = _s2s_call(h16, params["s2s_node"], NPK, None)
    ep = _s2s_call(e_in, params["s2s_edge"], S2S_EDGE_CHUNK, (wel, bel))
    return _pred_call(hp, ep, params["pred"])
